# baseline (speedup 1.0000x reference)
import functools

import jax
import jax.numpy as jnp
from jax import lax
from jax.experimental import pallas as pl
from jax.experimental.pallas import tpu as pltpu
from jax.experimental.pallas import tpu_sc as plsc

_N = 10000
_E = 320000
_IN = 128
_HID = 16
_OUT = 16
_HEADS = 8
_F1 = _HEADS * _HID
_C1 = _F1 + 16
_C2 = 32

_NTILES = 32
_EPT = _E // _NTILES
_EPT_P = 10240
_PAD_T = _EPT_P - _EPT
_TRASH = 64
_CH1 = 40
_CH2 = 128
_RPT = _N // 16
_ZCH = 125
_BN = 1000

def _lanes():
    return lax.iota(jnp.int32, 16)


_GDN = lax.GatherDimensionNumbers(
    offset_dims=(), collapsed_slice_dims=(0,), start_index_map=(0,))


def _lgather(v, idx):
    return lax.gather(v, idx[:, None], _GDN, (1,),
                      mode=lax.GatherScatterMode.PROMISE_IN_BOUNDS)


def _tc1_body(x_ref, w1_ref, b1m_ref, hcat_ref, ad_ref):
    h = jnp.dot(x_ref[...], w1_ref[...], preferred_element_type=jnp.float32)
    ad = jnp.dot(h, b1m_ref[...], preferred_element_type=jnp.float32)
    hcat_ref[:, :_F1] = h
    hcat_ref[:, _F1:] = ad
    ad_ref[...] = ad


_tc1 = pl.pallas_call(
    _tc1_body,
    grid=(_N // _BN,),
    in_specs=[
        pl.BlockSpec((_BN, _IN), lambda i: (i, 0)),
        pl.BlockSpec((_IN, _F1), lambda i: (0, 0)),
        pl.BlockSpec((_IN, 16), lambda i: (0, 0)),
    ],
    out_specs=[
        pl.BlockSpec((_BN, _C1), lambda i: (i, 0)),
        pl.BlockSpec((_BN, 16), lambda i: (i, 0)),
    ],
    out_shape=[
        jax.ShapeDtypeStruct((_N, _C1), jnp.float32),
        jax.ShapeDtypeStruct((_N, 16), jnp.float32),
    ],
)


def _make_sc_body(C, cn, make_consts, compute_edge):
    ns = _EPT_P // cn

    def body(hfeat_hbm, ad_hbm, ecp_hbm, out_hbm,
             ixb0, ixb1, ixb2, ixb3,
             rows0, rows1, outb0, outb1, adv0, adv1, shared,
             si0, si1, si2, si3, sr0, sr1, sa0, sa1, ssc0, ssc1):
        idxb = [ixb0, ixb1, ixb2, ixb3]
        rows = [rows0, rows1]
        outb = [outb0, outb1]
        adv = [adv0, adv1]
        semi = [si0, si1, si2, si3]
        semr = [sr0, sr1]
        sema = [sa0, sa1]
        semsc = [ssc0, ssc1]
        cc = lax.axis_index("c")
        s = lax.axis_index("s")
        consts = make_consts()
        zero16 = jnp.zeros((16,), jnp.float32)

        @plsc.parallel_loop(0, cn, unroll=8)
        def zrow(j):
            for k in range(C // 16):
                outb0[j, pl.ds(k * 16, 16)] = zero16

        r0 = s * _RPT
        nfull = _RPT // cn
        for i in range(nfull):
            pltpu.sync_copy(outb0, shared.at[pl.ds(r0 + i * cn, cn)])
        rem = _RPT - nfull * cn
        if rem:
            pltpu.sync_copy(outb0.at[pl.ds(0, rem)],
                            shared.at[pl.ds(r0 + _RPT - rem, rem)])
        plsc.subcore_barrier()

        cbase = (cc * 16 + s) * ns

        def start_idx(i, il):
            pltpu.async_copy(ecp_hbm.at[cbase + i], idxb[il], semi[il])

        def wait_idx(il):
            pltpu.make_async_copy(ecp_hbm.at[0], idxb[il], semi[il]).wait()

        def start_gather(il):
            d = il % 2
            pltpu.async_copy(hfeat_hbm.at[idxb[il].at[0]], rows[d], semr[d])
            pltpu.async_copy(ad_hbm.at[idxb[il].at[1]], adv[d], sema[d])

        def wait_gather(d):
            pltpu.make_async_copy(hfeat_hbm.at[pl.ds(0, cn)], rows[d],
                                  semr[d]).wait()
            pltpu.make_async_copy(ad_hbm.at[pl.ds(0, cn)], adv[d],
                                  sema[d]).wait()

        def start_scatter(d, il):
            pltpu.async_copy(outb[d], shared.at[idxb[il].at[2]], semsc[d],
                             add=True)

        def wait_scatter(d):
            pltpu.make_async_copy(outb[d], shared.at[pl.ds(0, cn)],
                                  semsc[d]).wait()

        def compute(d):
            ro, ao, ob = rows[d], adv[d], outb[d]

            @plsc.parallel_loop(0, cn, unroll=8)
            def edge(j):
                compute_edge(consts, ro, ao, ob, j)

        def phase(i, il, pre, ws, ii):
            d = il % 2
            if pre:
                jn = (il + 1) % 4
                wait_idx(jn)
                start_gather(jn)
            if ws:
                wait_scatter(d)
            wait_gather(d)
            if ii:
                start_idx(i + 2, (il + 2) % 4)
            compute(d)
            start_scatter(d, il)

        start_idx(0, 0)
        wait_idx(0)
        start_gather(0)
        start_idx(1, 1)
        phase(0, 0, True, False, True)
        phase(1, 1, True, False, True)

        tail = 4 + ((ns - 2) % 4)

        def quad(k, _):
            i = 4 * k + 2
            phase(i, 2, True, True, True)
            phase(i + 1, 3, True, True, True)
            phase(i + 2, 0, True, True, True)
            phase(i + 3, 1, True, True, True)
            return _

        lax.fori_loop(0, (ns - 2 - tail) // 4, quad, None)
        for i in range(ns - tail, ns):
            phase(i, i % 4, i + 1 < ns, True, i + 2 < ns)
        wait_scatter(0)
        wait_scatter(1)

        plsc.subcore_barrier()
        for i in range(_RPT // _ZCH):
            rr = r0 + i * _ZCH
            pltpu.sync_copy(shared.at[pl.ds(rr, _ZCH)],
                            out_hbm.at[cc, pl.ds(rr, _ZCH)])

    return body


def _sc_scratch(C, cn):
    return ([pltpu.VMEM((3, cn), jnp.int32)] * 4
            + [pltpu.VMEM((cn, C), jnp.float32)] * 4
            + [pltpu.VMEM((cn, 16), jnp.float32)] * 2
            + [pltpu.VMEM_SHARED((_N + _TRASH, C), jnp.float32)]
            + [pltpu.SemaphoreType.DMA] * 10)


def _consts1():
    return ((_lanes() + 8) & 15, _lanes() < 8)


def _edge1(consts, ro, ao, ob, j):
    perm, low = consts
    g8 = ro[j, pl.ds(_F1, 16)]
    d16 = ao[j, :]
    sv = g8 + _lgather(d16, perm)
    alpha = jnp.maximum(sv, 0.2 * sv)
    w = jnp.where(low, jnp.exp(alpha), 0.0)
    for h in range(_HEADS):
        sl = pl.ds(h * _HID, 16)
        ob[j, sl] = ro[j, sl] * _lgather(w, jnp.full((16,), h, jnp.int32))
    ob[j, pl.ds(_F1, 16)] = w


_sc1 = pl.kernel(
    _make_sc_body(_C1, _CH1, _consts1, _edge1),
    out_type=jax.ShapeDtypeStruct((2, _N, _C1), jnp.float32),
    mesh=plsc.VectorSubcoreMesh(core_axis_name="c", subcore_axis_name="s"),
    compiler_params=pltpu.CompilerParams(use_tc_tiling_on_sc=False),
    scratch_types=_sc_scratch(_C1, _CH1),
)


def _tc2_body(acc_ref, b1_ref, w2_ref, p_ref, q_ref, hcat2_ref, ad2_ref):
    acc = acc_ref[0] + acc_ref[1]
    parts = []
    for hd in range(_HEADS):
        dcol = acc[:, _F1 + hd:_F1 + hd + 1]
        parts.append(acc[:, hd * _HID:(hd + 1) * _HID] / (dcol + 1e-16))
    x2 = jnp.maximum(jnp.concatenate(parts, axis=1) + b1_ref[...], 0.0)
    h2 = jnp.dot(x2, w2_ref[...], preferred_element_type=jnp.float32)
    hcat2_ref[...] = jnp.dot(h2, p_ref[...], preferred_element_type=jnp.float32)
    ad2_ref[...] = jnp.dot(h2, q_ref[...], preferred_element_type=jnp.float32)


_tc2 = pl.pallas_call(
    _tc2_body,
    grid=(_N // _BN,),
    in_specs=[
        pl.BlockSpec((2, _BN, _C1), lambda i: (0, i, 0)),
        pl.BlockSpec((1, _F1), lambda i: (0, 0)),
        pl.BlockSpec((_F1, _OUT), lambda i: (0, 0)),
        pl.BlockSpec((_OUT, _C2), lambda i: (0, 0)),
        pl.BlockSpec((_OUT, 16), lambda i: (0, 0)),
    ],
    out_specs=[
        pl.BlockSpec((_BN, _C2), lambda i: (i, 0)),
        pl.BlockSpec((_BN, 16), lambda i: (i, 0)),
    ],
    out_shape=[
        jax.ShapeDtypeStruct((_N, _C2), jnp.float32),
        jax.ShapeDtypeStruct((_N, 16), jnp.float32),
    ],
)


def _consts2():
    return (_lanes() == 0, jnp.zeros((16,), jnp.int32),
            jnp.ones((16,), jnp.int32))


def _edge2(consts, ro, ao, ob, j):
    lane0, i0, i1 = consts
    g0 = ro[j, pl.ds(0, 16)]
    g1 = ro[j, pl.ds(16, 16)]
    d16 = ao[j, :]
    sv = _lgather(g1, i0) + _lgather(d16, i1)
    alpha = jnp.maximum(sv, 0.2 * sv)
    w = jnp.exp(alpha)
    ob[j, pl.ds(0, 16)] = g0 * w
    ob[j, pl.ds(16, 16)] = jnp.where(lane0, w, 0.0)


_sc2 = pl.kernel(
    _make_sc_body(_C2, _CH2, _consts2, _edge2),
    out_type=jax.ShapeDtypeStruct((2, _N, _C2), jnp.float32),
    mesh=plsc.VectorSubcoreMesh(core_axis_name="c", subcore_axis_name="s"),
    compiler_params=pltpu.CompilerParams(use_tc_tiling_on_sc=False),
    scratch_types=_sc_scratch(_C2, _CH2),
)


def _tc3_body(acc_ref, b2_ref, out_ref):
    acc = acc_ref[0] + acc_ref[1]
    v = acc[:, :_OUT] / (acc[:, _OUT:_OUT + 1] + 1e-16) + b2_ref[...]
    z = v - jnp.max(v, axis=1, keepdims=True)
    out_ref[...] = z - jnp.log(jnp.sum(jnp.exp(z), axis=1, keepdims=True))


_tc3 = pl.pallas_call(
    _tc3_body,
    grid=(_N // _BN,),
    in_specs=[
        pl.BlockSpec((2, _BN, _C2), lambda i: (0, i, 0)),
        pl.BlockSpec((1, _OUT), lambda i: (0, 0)),
    ],
    out_specs=pl.BlockSpec((_BN, _OUT), lambda i: (i, 0)),
    out_shape=jax.ShapeDtypeStruct((_N, _OUT), jnp.float32),
)


def _pack_edges(src, dst, chunk):
    ns = _EPT_P // chunk
    srcr = src.reshape(_NTILES, _EPT)
    dstr = dst.reshape(_NTILES, _EPT)
    zpad = jnp.zeros((_NTILES, _PAD_T), jnp.int32)
    tr = _N + (jnp.arange(_PAD_T, dtype=jnp.int32) % _TRASH)
    trpad = jnp.broadcast_to(tr, (_NTILES, _PAD_T))
    srcp = jnp.concatenate([srcr, zpad], axis=1)
    dstg = jnp.concatenate([dstr, zpad], axis=1)
    dsts = jnp.concatenate([dstr, trpad], axis=1)
    ecp = jnp.stack([srcp, dstg, dsts], axis=1)
    ecp = ecp.reshape(_NTILES, 3, ns, chunk).transpose(0, 2, 1, 3)
    return ecp.reshape(_NTILES * ns, 3, chunk)


def kernel(x, edge_index, W1, att_src1, att_dst1, b1, W2, att_src2, att_dst2,
           b2):
    src = edge_index[0].astype(jnp.int32)
    dst = edge_index[1].astype(jnp.int32)
    ecp1 = _pack_edges(src, dst, _CH1)
    ecp2 = _pack_edges(src, dst, _CH2)

    eye8 = jnp.eye(_HEADS, dtype=jnp.float32)
    bsrc = (att_src1[:, :, None] * eye8[:, None, :]).reshape(_F1, _HEADS)
    bdst = (att_dst1[:, :, None] * eye8[:, None, :]).reshape(_F1, _HEADS)
    b1m = jnp.concatenate([bsrc, bdst], axis=1)
    p = (jnp.zeros((_OUT, _C2), jnp.float32)
         .at[:, :_OUT].set(jnp.eye(_OUT, dtype=jnp.float32))
         .at[:, _OUT].set(att_src2[0])
         .at[:, _OUT + 1].set(att_dst2[0]))
    q = (jnp.zeros((_OUT, 16), jnp.float32)
         .at[:, 0].set(att_src2[0])
         .at[:, 1].set(att_dst2[0]))

    hcat, ad = _tc1(x, W1, b1m)
    acc1 = _sc1(hcat, ad, ecp1)
    hcat2, ad2 = _tc2(acc1, b1.reshape(1, _F1), W2, p, q)
    acc2 = _sc2(hcat2, ad2, ecp2)
    return _tc3(acc2, b2.reshape(1, _OUT))


# speedup vs baseline: 1.7376x; 1.7376x over previous
import functools

import jax
import jax.numpy as jnp
from jax import lax
from jax.experimental import pallas as pl
from jax.experimental.pallas import tpu as pltpu
from jax.experimental.pallas import tpu_sc as plsc

_N = 10000
_E = 320000
_IN = 128
_HID = 16
_OUT = 16
_HEADS = 8
_F1 = _HEADS * _HID
_C1 = _F1 + 16
_C2 = 32

_NTILES = 32
_EPT = _E // _NTILES
_EPT_P = 10240
_PAD_T = _EPT_P - _EPT
_TRASH = 64
_CH1 = 40
_CH2 = 128
_RPT = _N // 16
_ZCH = 125
_BN = 1000

def _lanes():
    return lax.iota(jnp.int32, 16)


_GDN = lax.GatherDimensionNumbers(
    offset_dims=(), collapsed_slice_dims=(0,), start_index_map=(0,))


def _lgather(v, idx):
    return lax.gather(v, idx[:, None], _GDN, (1,),
                      mode=lax.GatherScatterMode.PROMISE_IN_BOUNDS)


def _tc1_body(x_ref, w1_ref, b1m_ref, hcat_ref, ad_ref):
    h = jnp.dot(x_ref[...], w1_ref[...], preferred_element_type=jnp.float32)
    ad = jnp.dot(h, b1m_ref[...], preferred_element_type=jnp.float32)
    hcat_ref[:, :_F1] = h
    hcat_ref[:, _F1:] = ad
    ad_ref[...] = ad


_tc1 = pl.pallas_call(
    _tc1_body,
    grid=(_N // _BN,),
    in_specs=[
        pl.BlockSpec((_BN, _IN), lambda i: (i, 0)),
        pl.BlockSpec((_IN, _F1), lambda i: (0, 0)),
        pl.BlockSpec((_IN, 16), lambda i: (0, 0)),
    ],
    out_specs=[
        pl.BlockSpec((_BN, _C1), lambda i: (i, 0)),
        pl.BlockSpec((_BN, 16), lambda i: (i, 0)),
    ],
    out_shape=[
        jax.ShapeDtypeStruct((_N, _C1), jnp.float32),
        jax.ShapeDtypeStruct((_N, 16), jnp.float32),
    ],
)


def _make_sc_body(C, cn, make_consts, compute_edge):
    ns = _EPT_P // cn

    def body(hfeat_hbm, ad_hbm, ecp_hbm, out_hbm,
             ixb0, ixb1, ixb2, ixb3,
             rows0, rows1, outb0, outb1, adv0, adv1, shared,
             si0, si1, si2, si3, sr0, sr1, sa0, sa1, ssc0, ssc1):
        idxb = [ixb0, ixb1, ixb2, ixb3]
        rows = [rows0, rows1]
        outb = [outb0, outb1]
        adv = [adv0, adv1]
        semi = [si0, si1, si2, si3]
        semr = [sr0, sr1]
        sema = [sa0, sa1]
        semsc = [ssc0, ssc1]
        cc = lax.axis_index("c")
        s = lax.axis_index("s")
        consts = make_consts()
        zero16 = jnp.zeros((16,), jnp.float32)

        @plsc.parallel_loop(0, cn, unroll=8)
        def zrow(j):
            for k in range(C // 16):
                outb0[j, pl.ds(k * 16, 16)] = zero16

        r0 = s * _RPT
        nfull = _RPT // cn
        for i in range(nfull):
            pltpu.sync_copy(outb0, shared.at[pl.ds(r0 + i * cn, cn)])
        rem = _RPT - nfull * cn
        if rem:
            pltpu.sync_copy(outb0.at[pl.ds(0, rem)],
                            shared.at[pl.ds(r0 + _RPT - rem, rem)])
        plsc.subcore_barrier()

        cbase = (cc * 16 + s) * ns

        def start_idx(i, il):
            pltpu.async_copy(ecp_hbm.at[cbase + i], idxb[il], semi[il])

        def wait_idx(il):
            pltpu.make_async_copy(ecp_hbm.at[0], idxb[il], semi[il]).wait()

        def start_gather(il):
            d = il % 2
            pltpu.async_copy(hfeat_hbm.at[idxb[il].at[0]], rows[d], semr[d])
            pltpu.async_copy(ad_hbm.at[idxb[il].at[1]], adv[d], sema[d])

        def wait_gather(d):
            pltpu.make_async_copy(hfeat_hbm.at[pl.ds(0, cn)], rows[d],
                                  semr[d]).wait()
            pltpu.make_async_copy(ad_hbm.at[pl.ds(0, cn)], adv[d],
                                  sema[d]).wait()

        def start_scatter(d, il):
            pltpu.async_copy(outb[d], shared.at[idxb[il].at[2]], semsc[d],
                             add=True)

        def wait_scatter(d):
            pltpu.make_async_copy(outb[d], shared.at[pl.ds(0, cn)],
                                  semsc[d]).wait()

        def compute(d):
            ro, ao, ob = rows[d], adv[d], outb[d]

            @plsc.parallel_loop(0, cn, unroll=8)
            def edge(j):
                compute_edge(consts, ro, ao, ob, j)

        def phase(i, il, pre, ws, ii):
            d = il % 2
            if pre:
                jn = (il + 1) % 4
                wait_idx(jn)
                start_gather(jn)
            if ws:
                wait_scatter(d)
            wait_gather(d)
            if ii:
                start_idx(i + 2, (il + 2) % 4)
            compute(d)
            start_scatter(d, il)

        start_idx(0, 0)
        wait_idx(0)
        start_gather(0)
        start_idx(1, 1)
        phase(0, 0, True, False, True)
        phase(1, 1, True, False, True)

        tail = 4 + ((ns - 2) % 4)

        def quad(k, _):
            i = 4 * k + 2
            phase(i, 2, True, True, True)
            phase(i + 1, 3, True, True, True)
            phase(i + 2, 0, True, True, True)
            phase(i + 3, 1, True, True, True)
            return _

        lax.fori_loop(0, (ns - 2 - tail) // 4, quad, None)
        for i in range(ns - tail, ns):
            phase(i, i % 4, i + 1 < ns, True, i + 2 < ns)
        wait_scatter(0)
        wait_scatter(1)

        plsc.subcore_barrier()
        for i in range(_RPT // _ZCH):
            rr = r0 + i * _ZCH
            pltpu.sync_copy(shared.at[pl.ds(rr, _ZCH)],
                            out_hbm.at[cc, pl.ds(rr, _ZCH)])

    return body


def _sc_scratch(C, cn):
    return ([pltpu.VMEM((3, cn), jnp.int32)] * 4
            + [pltpu.VMEM((cn, C), jnp.float32)] * 4
            + [pltpu.VMEM((cn, 16), jnp.float32)] * 2
            + [pltpu.VMEM_SHARED((_N + _TRASH, C), jnp.float32)]
            + [pltpu.SemaphoreType.DMA] * 10)


def _make_sc_body_sep(C, cn, make_consts, compute_edge):
    ns = _EPT // cn

    def body(hfeat_hbm, ad_hbm, src_hbm, dst_hbm, out_hbm,
             ixs0, ixs1, ixs2, ixs3, ixd0, ixd1, ixd2, ixd3,
             rows0, rows1, outb0, outb1, adv0, adv1, shared,
             sis0, sis1, sis2, sis3, sid0, sid1, sid2, sid3,
             sr0, sr1, sa0, sa1, ssc0, ssc1):
        idx_s = [ixs0, ixs1, ixs2, ixs3]
        idx_d = [ixd0, ixd1, ixd2, ixd3]
        rows = [rows0, rows1]
        outb = [outb0, outb1]
        adv = [adv0, adv1]
        semis = [sis0, sis1, sis2, sis3]
        semid = [sid0, sid1, sid2, sid3]
        semr = [sr0, sr1]
        sema = [sa0, sa1]
        semsc = [ssc0, ssc1]
        cc = lax.axis_index("c")
        s = lax.axis_index("s")
        consts = make_consts()
        zero16 = jnp.zeros((16,), jnp.float32)

        @plsc.parallel_loop(0, cn, unroll=8)
        def zrow(j):
            for k in range(C // 16):
                outb0[j, pl.ds(k * 16, 16)] = zero16

        r0 = s * _RPT
        nfull = _RPT // cn
        for i in range(nfull):
            pltpu.sync_copy(outb0, shared.at[pl.ds(r0 + i * cn, cn)])
        rem = _RPT - nfull * cn
        if rem:
            pltpu.sync_copy(outb0.at[pl.ds(0, rem)],
                            shared.at[pl.ds(r0 + _RPT - rem, rem)])
        plsc.subcore_barrier()

        ebase = (cc * 16 + s) * _EPT

        def start_idx(i, il):
            b = pl.multiple_of(ebase + i * cn, 8)
            pltpu.async_copy(src_hbm.at[pl.ds(b, cn)], idx_s[il], semis[il])
            pltpu.async_copy(dst_hbm.at[pl.ds(b, cn)], idx_d[il], semid[il])

        def wait_idx(il):
            pltpu.make_async_copy(src_hbm.at[pl.ds(0, cn)], idx_s[il],
                                  semis[il]).wait()
            pltpu.make_async_copy(dst_hbm.at[pl.ds(0, cn)], idx_d[il],
                                  semid[il]).wait()

        def start_gather(il):
            d = il % 2
            pltpu.async_copy(hfeat_hbm.at[idx_s[il]], rows[d], semr[d])
            pltpu.async_copy(ad_hbm.at[idx_d[il]], adv[d], sema[d])

        def wait_gather(d):
            pltpu.make_async_copy(hfeat_hbm.at[pl.ds(0, cn)], rows[d],
                                  semr[d]).wait()
            pltpu.make_async_copy(ad_hbm.at[pl.ds(0, cn)], adv[d],
                                  sema[d]).wait()

        def start_scatter(d, il):
            pltpu.async_copy(outb[d], shared.at[idx_d[il]], semsc[d],
                             add=True)

        def wait_scatter(d):
            pltpu.make_async_copy(outb[d], shared.at[pl.ds(0, cn)],
                                  semsc[d]).wait()

        def compute(d):
            ro, ao, ob = rows[d], adv[d], outb[d]

            @plsc.parallel_loop(0, cn, unroll=8)
            def edge(j):
                compute_edge(consts, ro, ao, ob, j)

        def phase(i, il, pre, ws, ii):
            d = il % 2
            if pre:
                jn = (il + 1) % 4
                wait_idx(jn)
                start_gather(jn)
            if ws:
                wait_scatter(d)
            wait_gather(d)
            if ii:
                start_idx(i + 2, (il + 2) % 4)
            compute(d)
            start_scatter(d, il)

        start_idx(0, 0)
        wait_idx(0)
        start_gather(0)
        start_idx(1, 1)
        phase(0, 0, True, False, True)
        phase(1, 1, True, False, True)

        tail = 4 + ((ns - 2) % 4)

        def quad(k, _):
            i = 4 * k + 2
            phase(i, 2, True, True, True)
            phase(i + 1, 3, True, True, True)
            phase(i + 2, 0, True, True, True)
            phase(i + 3, 1, True, True, True)
            return _

        lax.fori_loop(0, (ns - 2 - tail) // 4, quad, None)
        for i in range(ns - tail, ns):
            phase(i, i % 4, i + 1 < ns, True, i + 2 < ns)
        wait_scatter(0)
        wait_scatter(1)

        plsc.subcore_barrier()
        for i in range(_RPT // _ZCH):
            rr = r0 + i * _ZCH
            pltpu.sync_copy(shared.at[pl.ds(rr, _ZCH)],
                            out_hbm.at[cc, pl.ds(rr, _ZCH)])

    return body


def _sc_scratch_sep(C, cn):
    return ([pltpu.VMEM((cn,), jnp.int32)] * 8
            + [pltpu.VMEM((cn, C), jnp.float32)] * 4
            + [pltpu.VMEM((cn, 16), jnp.float32)] * 2
            + [pltpu.VMEM_SHARED((_N, C), jnp.float32)]
            + [pltpu.SemaphoreType.DMA] * 14)


def _consts1():
    return ((_lanes() + 8) & 15, _lanes() < 8)


def _edge1(consts, ro, ao, ob, j):
    perm, low = consts
    g8 = ro[j, pl.ds(_F1, 16)]
    d16 = ao[j, :]
    sv = g8 + _lgather(d16, perm)
    alpha = jnp.maximum(sv, 0.2 * sv)
    w = jnp.where(low, jnp.exp(alpha), 0.0)
    for h in range(_HEADS):
        sl = pl.ds(h * _HID, 16)
        ob[j, sl] = ro[j, sl] * _lgather(w, jnp.full((16,), h, jnp.int32))
    ob[j, pl.ds(_F1, 16)] = w


_sc1 = pl.kernel(
    _make_sc_body_sep(_C1, _CH1, _consts1, _edge1),
    out_type=jax.ShapeDtypeStruct((2, _N, _C1), jnp.float32),
    mesh=plsc.VectorSubcoreMesh(core_axis_name="c", subcore_axis_name="s"),
    compiler_params=pltpu.CompilerParams(use_tc_tiling_on_sc=False),
    scratch_types=_sc_scratch_sep(_C1, _CH1),
)


def _tc2_body(acc_ref, b1_ref, w2_ref, p_ref, q_ref, hcat2_ref, ad2_ref):
    acc = acc_ref[0] + acc_ref[1]
    parts = []
    for hd in range(_HEADS):
        dcol = acc[:, _F1 + hd:_F1 + hd + 1]
        parts.append(acc[:, hd * _HID:(hd + 1) * _HID] / (dcol + 1e-16))
    x2 = jnp.maximum(jnp.concatenate(parts, axis=1) + b1_ref[...], 0.0)
    h2 = jnp.dot(x2, w2_ref[...], preferred_element_type=jnp.float32)
    hcat2_ref[...] = jnp.dot(h2, p_ref[...], preferred_element_type=jnp.float32)
    ad2_ref[...] = jnp.dot(h2, q_ref[...], preferred_element_type=jnp.float32)


_tc2 = pl.pallas_call(
    _tc2_body,
    grid=(_N // _BN,),
    in_specs=[
        pl.BlockSpec((2, _BN, _C1), lambda i: (0, i, 0)),
        pl.BlockSpec((1, _F1), lambda i: (0, 0)),
        pl.BlockSpec((_F1, _OUT), lambda i: (0, 0)),
        pl.BlockSpec((_OUT, _C2), lambda i: (0, 0)),
        pl.BlockSpec((_OUT, 16), lambda i: (0, 0)),
    ],
    out_specs=[
        pl.BlockSpec((_BN, _C2), lambda i: (i, 0)),
        pl.BlockSpec((_BN, 16), lambda i: (i, 0)),
    ],
    out_shape=[
        jax.ShapeDtypeStruct((_N, _C2), jnp.float32),
        jax.ShapeDtypeStruct((_N, 16), jnp.float32),
    ],
)


def _consts2():
    return (_lanes() == 0, jnp.zeros((16,), jnp.int32),
            jnp.ones((16,), jnp.int32))


def _edge2(consts, ro, ao, ob, j):
    lane0, i0, i1 = consts
    g0 = ro[j, pl.ds(0, 16)]
    g1 = ro[j, pl.ds(16, 16)]
    d16 = ao[j, :]
    sv = _lgather(g1, i0) + _lgather(d16, i1)
    alpha = jnp.maximum(sv, 0.2 * sv)
    w = jnp.exp(alpha)
    ob[j, pl.ds(0, 16)] = g0 * w
    ob[j, pl.ds(16, 16)] = jnp.where(lane0, w, 0.0)


_sc2 = pl.kernel(
    _make_sc_body(_C2, _CH2, _consts2, _edge2),
    out_type=jax.ShapeDtypeStruct((2, _N, _C2), jnp.float32),
    mesh=plsc.VectorSubcoreMesh(core_axis_name="c", subcore_axis_name="s"),
    compiler_params=pltpu.CompilerParams(use_tc_tiling_on_sc=False),
    scratch_types=_sc_scratch(_C2, _CH2),
)


def _tc3_body(acc_ref, b2_ref, out_ref):
    acc = acc_ref[0] + acc_ref[1]
    v = acc[:, :_OUT] / (acc[:, _OUT:_OUT + 1] + 1e-16) + b2_ref[...]
    z = v - jnp.max(v, axis=1, keepdims=True)
    out_ref[...] = z - jnp.log(jnp.sum(jnp.exp(z), axis=1, keepdims=True))


_tc3 = pl.pallas_call(
    _tc3_body,
    grid=(_N // _BN,),
    in_specs=[
        pl.BlockSpec((2, _BN, _C2), lambda i: (0, i, 0)),
        pl.BlockSpec((1, _OUT), lambda i: (0, 0)),
    ],
    out_specs=pl.BlockSpec((_BN, _OUT), lambda i: (i, 0)),
    out_shape=jax.ShapeDtypeStruct((_N, _OUT), jnp.float32),
)


def _pack_edges(src, dst, chunk):
    ns = _EPT_P // chunk
    srcr = src.reshape(_NTILES, _EPT)
    dstr = dst.reshape(_NTILES, _EPT)
    zpad = jnp.zeros((_NTILES, _PAD_T), jnp.int32)
    tr = _N + (jnp.arange(_PAD_T, dtype=jnp.int32) % _TRASH)
    trpad = jnp.broadcast_to(tr, (_NTILES, _PAD_T))
    srcp = jnp.concatenate([srcr, zpad], axis=1)
    dstg = jnp.concatenate([dstr, zpad], axis=1)
    dsts = jnp.concatenate([dstr, trpad], axis=1)
    ecp = jnp.stack([srcp, dstg, dsts], axis=1)
    ecp = ecp.reshape(_NTILES, 3, ns, chunk).transpose(0, 2, 1, 3)
    return ecp.reshape(_NTILES * ns, 3, chunk)


def kernel(x, edge_index, W1, att_src1, att_dst1, b1, W2, att_src2, att_dst2,
           b2):
    src = edge_index[0].astype(jnp.int32)
    dst = edge_index[1].astype(jnp.int32)
    ecp2 = _pack_edges(src, dst, _CH2)

    eye8 = jnp.eye(_HEADS, dtype=jnp.float32)
    bsrc = (att_src1[:, :, None] * eye8[:, None, :]).reshape(_F1, _HEADS)
    bdst = (att_dst1[:, :, None] * eye8[:, None, :]).reshape(_F1, _HEADS)
    b1m = jnp.concatenate([bsrc, bdst], axis=1)
    p = (jnp.zeros((_OUT, _C2), jnp.float32)
         .at[:, :_OUT].set(jnp.eye(_OUT, dtype=jnp.float32))
         .at[:, _OUT].set(att_src2[0])
         .at[:, _OUT + 1].set(att_dst2[0]))
    q = (jnp.zeros((_OUT, 16), jnp.float32)
         .at[:, 0].set(att_src2[0])
         .at[:, 1].set(att_dst2[0]))

    hcat, ad = _tc1(x, W1, b1m)
    acc1 = _sc1(hcat, ad, src, dst)
    hcat2, ad2 = _tc2(acc1, b1.reshape(1, _F1), W2, p, q)
    acc2 = _sc2(hcat2, ad2, ecp2)
    return _tc3(acc2, b2.reshape(1, _OUT))
